# baseline (speedup 1.0000x reference)
import functools

import jax
import jax.numpy as jnp
from jax import lax
from jax.experimental import pallas as pl
from jax.experimental.pallas import tpu as pltpu
from jax.experimental.pallas import tpu_sc as plsc

N_IDX = 3276800
DIM_EMB = 16

_info = plsc.get_sparse_core_info()
_NC, _NS = _info.num_cores, _info.num_subcores
_NW = _NC * _NS

_CHUNK = 1024
_BLOCK = _NS * _CHUNK
_NBLOCKS = N_IDX // _BLOCK
_ROUNDS = _NBLOCKS // _NC
_NBUF = 2


def _gather_kernel(idx_hbm, table_hbm, out_hbm, idx_v, rows_v, stage_sh,
                   sem_idx, sem_g, sem_o):
    cid = lax.axis_index("c")
    sid = lax.axis_index("s")

    def chunk_off(r):
        return (r * _NC + cid) * _BLOCK + sid * _CHUNK

    for b in range(_NBUF):
        pltpu.sync_copy(idx_hbm.at[pl.ds(chunk_off(b), _CHUNK)], idx_v.at[b])
        pltpu.async_copy(table_hbm.at[idx_v.at[b]], rows_v.at[b], sem_g.at[b])

    def body(j, carry):
        for b in range(_NBUF):
            r = j * _NBUF + b

            pltpu.make_async_copy(
                table_hbm.at[idx_v.at[b]], rows_v.at[b], sem_g.at[b]).wait()

            @pl.when(r + _NBUF < _ROUNDS)
            def _():
                pltpu.async_copy(
                    idx_hbm.at[pl.ds(chunk_off(r + _NBUF), _CHUNK)],
                    idx_v.at[b], sem_idx.at[b])

            @pl.when(jnp.logical_and(j > 0, sid == 0))
            def _():
                pltpu.make_async_copy(
                    stage_sh.at[b],
                    out_hbm.at[pl.ds(((r - _NBUF) * _NC + cid) * _BLOCK,
                                     _BLOCK)],
                    sem_o.at[b]).wait()
            plsc.subcore_barrier()

            pltpu.sync_copy(rows_v.at[b],
                            stage_sh.at[b, pl.ds(sid * _CHUNK, _CHUNK)])

            @pl.when(r + _NBUF < _ROUNDS)
            def _():
                pltpu.make_async_copy(
                    idx_hbm.at[pl.ds(chunk_off(r + _NBUF), _CHUNK)],
                    idx_v.at[b], sem_idx.at[b]).wait()
                pltpu.async_copy(
                    table_hbm.at[idx_v.at[b]], rows_v.at[b], sem_g.at[b])

            plsc.subcore_barrier()

            @pl.when(sid == 0)
            def _():
                pltpu.async_copy(
                    stage_sh.at[b],
                    out_hbm.at[pl.ds((r * _NC + cid) * _BLOCK, _BLOCK)],
                    sem_o.at[b])
        return carry

    lax.fori_loop(0, _ROUNDS // _NBUF, body, 0)

    @pl.when(sid == 0)
    def _():
        for b in range(_NBUF):
            r = _ROUNDS - _NBUF + b
            pltpu.make_async_copy(
                stage_sh.at[b],
                out_hbm.at[pl.ds((r * _NC + cid) * _BLOCK, _BLOCK)],
                sem_o.at[b]).wait()


def kernel(node_classes, embedding):
    mesh = plsc.VectorSubcoreMesh(core_axis_name="c", subcore_axis_name="s")
    run = functools.partial(
        pl.kernel,
        mesh=mesh,
        out_type=jax.ShapeDtypeStruct((N_IDX, DIM_EMB), jnp.float32),
        scratch_types=[
            pltpu.VMEM((_NBUF, _CHUNK), jnp.int32),
            pltpu.VMEM((_NBUF, _CHUNK, DIM_EMB), jnp.float32),
            pltpu.VMEM_SHARED((_NBUF, _BLOCK, DIM_EMB), jnp.float32),
            pltpu.SemaphoreType.DMA((_NBUF,)),
            pltpu.SemaphoreType.DMA((_NBUF,)),
            pltpu.SemaphoreType.DMA((_NBUF,)),
        ],
        compiler_params=pltpu.CompilerParams(use_tc_tiling_on_sc=False),
    )(_gather_kernel)
    return run(node_classes.astype(jnp.int32), embedding)
